# baseline (speedup 1.0000x reference)
import numpy as np
import jax
import jax.numpy as jnp
from jax import lax
from jax.experimental import pallas as pl
from jax.experimental.pallas import tpu as pltpu, tpu_sc as plsc

NEG = 8192
RANGE_MAX = 1000000

_ROT_A = (13, 15, 26, 6)
_ROT_B = (17, 29, 16, 24)


def _np_threefry2x32(ks0, ks1, x0, x1):
    x0 = x0.astype(np.uint32)
    x1 = x1.astype(np.uint32)
    ks = [np.uint32(ks0), np.uint32(ks1),
          np.uint32(np.uint32(ks0) ^ np.uint32(ks1) ^ np.uint32(0x1BD11BDA))]
    rot = [_ROT_A, _ROT_B]
    x0 = (x0 + ks[0]).astype(np.uint32)
    x1 = (x1 + ks[1]).astype(np.uint32)
    for i in range(5):
        for r in rot[i % 2]:
            x0 = (x0 + x1).astype(np.uint32)
            x1 = ((x1 << np.uint32(r)) | (x1 >> np.uint32(32 - r))).astype(np.uint32)
            x1 = (x0 ^ x1).astype(np.uint32)
        x0 = (x0 + ks[(i + 1) % 3]).astype(np.uint32)
        x1 = (x1 + ks[(i + 2) % 3] + np.uint32(i + 1)).astype(np.uint32)
    return x0, x1


_o0, _o1 = _np_threefry2x32(0, 0, np.array([0], np.uint32), np.array([1234], np.uint32))
_KS0 = int(_o0[0])
_KS1 = int(_o1[0])
_KS2 = _KS0 ^ _KS1 ^ 0x1BD11BDA
_LOG_RANGE = float(np.log(np.float32(RANGE_MAX + 1.0)))

_info = plsc.get_sparse_core_info()
_NS, _L = _info.num_subcores, _info.num_lanes
_NC = 1
_NW = _NC * _NS
_PER_W = NEG // _NW
_VECS = _PER_W // _L


def _sc_body(out_hbm, buf_v):
    wid = lax.axis_index("s") * _NC + lax.axis_index("c")
    base = wid * _PER_W

    ks0 = jnp.uint32(_KS0)
    ks1 = jnp.uint32(_KS1)
    ks2 = jnp.uint32(_KS2)
    ks = (ks0, ks1, ks2)
    iota = lax.iota(jnp.uint32, _L)
    base_u = lax.convert_element_type(base, jnp.uint32)

    for v in range(_VECS):
        x1 = iota + (base_u + jnp.uint32(v * _L))
        x0 = jnp.full((_L,), jnp.uint32((_KS0) & 0xFFFFFFFF), jnp.uint32)
        x1 = x1 + ks1
        for i in range(5):
            for r in (_ROT_A if i % 2 == 0 else _ROT_B):
                x0 = x0 + x1
                x1 = (x1 << jnp.uint32(r)) | (x1 >> jnp.uint32(32 - r))
                x1 = x0 ^ x1
            x0 = x0 + ks[(i + 1) % 3]
            x1 = x1 + ks[(i + 2) % 3] + jnp.uint32(i + 1)
        bits = x0 ^ x1
        fb = (bits >> jnp.uint32(9)) | jnp.uint32(0x3F800000)
        u = lax.bitcast_convert_type(fb, jnp.float32) - jnp.float32(1.0)
        g = jnp.exp(u * jnp.float32(_LOG_RANGE))
        s = lax.convert_element_type(g, jnp.int32) - jnp.int32(1)
        s = jnp.minimum(jnp.maximum(s, jnp.int32(0)), jnp.int32(RANGE_MAX - 1))
        buf_v[pl.ds(v * _L, _L)] = s

    pltpu.sync_copy(buf_v, out_hbm.at[pl.ds(base, _PER_W)])


def _sampler():
    mesh = plsc.VectorSubcoreMesh(core_axis_name="c", subcore_axis_name="s",
                                  num_cores=_NC)
    return pl.kernel(
        _sc_body,
        out_type=jax.ShapeDtypeStruct((NEG,), jnp.int32),
        mesh=mesh,
        scratch_types=[pltpu.VMEM((_PER_W,), jnp.int32)],
    )


def kernel(inputs):
    del inputs
    return _sampler()()


# speedup vs baseline: 1.0818x; 1.0818x over previous
import numpy as np
import jax
import jax.numpy as jnp
from jax import lax
from jax.experimental import pallas as pl
from jax.experimental.pallas import tpu as pltpu, tpu_sc as plsc

NEG = 8192
RANGE_MAX = 1000000

_ROT_A = (13, 15, 26, 6)
_ROT_B = (17, 29, 16, 24)


def _np_threefry2x32(ks0, ks1, x0, x1):
    x0 = x0.astype(np.uint32)
    x1 = x1.astype(np.uint32)
    ks = [np.uint32(ks0), np.uint32(ks1),
          np.uint32(np.uint32(ks0) ^ np.uint32(ks1) ^ np.uint32(0x1BD11BDA))]
    rot = [_ROT_A, _ROT_B]
    x0 = (x0 + ks[0]).astype(np.uint32)
    x1 = (x1 + ks[1]).astype(np.uint32)
    for i in range(5):
        for r in rot[i % 2]:
            x0 = (x0 + x1).astype(np.uint32)
            x1 = ((x1 << np.uint32(r)) | (x1 >> np.uint32(32 - r))).astype(np.uint32)
            x1 = (x0 ^ x1).astype(np.uint32)
        x0 = (x0 + ks[(i + 1) % 3]).astype(np.uint32)
        x1 = (x1 + ks[(i + 2) % 3] + np.uint32(i + 1)).astype(np.uint32)
    return x0, x1


_o0, _o1 = _np_threefry2x32(0, 0, np.array([0], np.uint32), np.array([1234], np.uint32))
_KS0 = int(_o0[0])
_KS1 = int(_o1[0])
_KS2 = _KS0 ^ _KS1 ^ 0x1BD11BDA
_LOG_RANGE = float(np.log(np.float32(RANGE_MAX + 1.0)))

_info = plsc.get_sparse_core_info()
_NS, _L = _info.num_subcores, _info.num_lanes
_NC = _info.num_cores
_NW = _NC * _NS
_PER_W = NEG // _NW
_VECS = _PER_W // _L


def _sc_body(out_hbm, buf_v):
    wid = lax.axis_index("s") * _NC + lax.axis_index("c")
    base = wid * _PER_W

    ks0 = jnp.uint32(_KS0)
    ks1 = jnp.uint32(_KS1)
    ks2 = jnp.uint32(_KS2)
    ks = (ks0, ks1, ks2)
    iota = lax.iota(jnp.uint32, _L)
    base_u = lax.convert_element_type(base, jnp.uint32)

    def body(v, _):
        x1 = iota + (base_u + lax.convert_element_type(v, jnp.uint32) * jnp.uint32(_L))
        x0 = jnp.full((_L,), jnp.uint32(_KS0), jnp.uint32)
        x1 = x1 + ks1
        for i in range(5):
            for r in (_ROT_A if i % 2 == 0 else _ROT_B):
                x0 = x0 + x1
                x1 = (x1 << jnp.uint32(r)) | (x1 >> jnp.uint32(32 - r))
                x1 = x0 ^ x1
            x0 = x0 + ks[(i + 1) % 3]
            x1 = x1 + ks[(i + 2) % 3] + jnp.uint32(i + 1)
        bits = x0 ^ x1
        fb = (bits >> jnp.uint32(9)) | jnp.uint32(0x3F800000)
        u = lax.bitcast_convert_type(fb, jnp.float32) - jnp.float32(1.0)
        g = jnp.exp(u * jnp.float32(_LOG_RANGE))
        s = lax.convert_element_type(g, jnp.int32) - jnp.int32(1)
        s = jnp.minimum(jnp.maximum(s, jnp.int32(0)), jnp.int32(RANGE_MAX - 1))
        buf_v[pl.ds(v * _L, _L)] = s
        return ()

    lax.fori_loop(0, _VECS, body, ())

    pltpu.sync_copy(buf_v, out_hbm.at[pl.ds(base, _PER_W)])


def _sampler():
    mesh = plsc.VectorSubcoreMesh(core_axis_name="c", subcore_axis_name="s",
                                  num_cores=_NC)
    return pl.kernel(
        _sc_body,
        out_type=jax.ShapeDtypeStruct((NEG,), jnp.int32),
        mesh=mesh,
        scratch_types=[pltpu.VMEM((_PER_W,), jnp.int32)],
    )


def kernel(inputs):
    del inputs
    return _sampler()()


# speedup vs baseline: 9.7238x; 8.9886x over previous
import numpy as np
import jax
import jax.numpy as jnp
from jax import lax
from jax.experimental import pallas as pl

NEG = 8192
RANGE_MAX = 1000000

_ROT_A = (13, 15, 26, 6)
_ROT_B = (17, 29, 16, 24)


def _np_threefry2x32(ks0, ks1, x0, x1):
    x0 = x0.astype(np.uint32)
    x1 = x1.astype(np.uint32)
    ks = [np.uint32(ks0), np.uint32(ks1),
          np.uint32(np.uint32(ks0) ^ np.uint32(ks1) ^ np.uint32(0x1BD11BDA))]
    rot = [_ROT_A, _ROT_B]
    x0 = (x0 + ks[0]).astype(np.uint32)
    x1 = (x1 + ks[1]).astype(np.uint32)
    for i in range(5):
        for r in rot[i % 2]:
            x0 = (x0 + x1).astype(np.uint32)
            x1 = ((x1 << np.uint32(r)) | (x1 >> np.uint32(32 - r))).astype(np.uint32)
            x1 = (x0 ^ x1).astype(np.uint32)
        x0 = (x0 + ks[(i + 1) % 3]).astype(np.uint32)
        x1 = (x1 + ks[(i + 2) % 3] + np.uint32(i + 1)).astype(np.uint32)
    return x0, x1


_o0, _o1 = _np_threefry2x32(0, 0, np.array([0], np.uint32), np.array([1234], np.uint32))
_KS0 = int(_o0[0])
_KS1 = int(_o1[0])
_KS2 = _KS0 ^ _KS1 ^ 0x1BD11BDA
_LOG_RANGE = float(np.log(np.float32(RANGE_MAX + 1.0)))

_R, _C = 8, 1024


def _tc_body(out_ref):
    ks = (jnp.uint32(_KS0), jnp.uint32(_KS1), jnp.uint32(_KS2))
    idx = (lax.broadcasted_iota(jnp.uint32, (_R, _C), 0) * jnp.uint32(_C)
           + lax.broadcasted_iota(jnp.uint32, (_R, _C), 1))
    x0 = jnp.full((_R, _C), jnp.uint32(_KS0), jnp.uint32)
    x1 = idx + ks[1]
    for i in range(5):
        for r in (_ROT_A if i % 2 == 0 else _ROT_B):
            x0 = x0 + x1
            x1 = (x1 << jnp.uint32(r)) | (x1 >> jnp.uint32(32 - r))
            x1 = x0 ^ x1
        x0 = x0 + ks[(i + 1) % 3]
        x1 = x1 + ks[(i + 2) % 3] + jnp.uint32(i + 1)
    bits = x0 ^ x1
    fb = (bits >> jnp.uint32(9)) | jnp.uint32(0x3F800000)
    u = lax.bitcast_convert_type(fb, jnp.float32) - jnp.float32(1.0)
    g = jnp.exp(u * jnp.float32(_LOG_RANGE))
    s = lax.convert_element_type(g, jnp.int32) - jnp.int32(1)
    out_ref[...] = jnp.minimum(jnp.maximum(s, jnp.int32(0)), jnp.int32(RANGE_MAX - 1))


def kernel(inputs):
    del inputs
    out = pl.pallas_call(
        _tc_body,
        out_shape=jax.ShapeDtypeStruct((_R, _C), jnp.int32),
    )()
    return out.reshape(NEG)


# speedup vs baseline: 26.7957x; 2.7557x over previous
import numpy as np
import jax
import jax.numpy as jnp
from jax import lax
from jax.experimental import pallas as pl

NEG = 8192
RANGE_MAX = 1000000

_ROT_A = (13, 15, 26, 6)
_ROT_B = (17, 29, 16, 24)


def _np_threefry2x32(ks0, ks1, x0, x1):
    x0 = x0.astype(np.uint32)
    x1 = x1.astype(np.uint32)
    ks = [np.uint32(ks0), np.uint32(ks1),
          np.uint32(np.uint32(ks0) ^ np.uint32(ks1) ^ np.uint32(0x1BD11BDA))]
    rot = [_ROT_A, _ROT_B]
    x0 = (x0 + ks[0]).astype(np.uint32)
    x1 = (x1 + ks[1]).astype(np.uint32)
    for i in range(5):
        for r in rot[i % 2]:
            x0 = (x0 + x1).astype(np.uint32)
            x1 = ((x1 << np.uint32(r)) | (x1 >> np.uint32(32 - r))).astype(np.uint32)
            x1 = (x0 ^ x1).astype(np.uint32)
        x0 = (x0 + ks[(i + 1) % 3]).astype(np.uint32)
        x1 = (x1 + ks[(i + 2) % 3] + np.uint32(i + 1)).astype(np.uint32)
    return x0, x1


_o0, _o1 = _np_threefry2x32(0, 0, np.array([0], np.uint32), np.array([1234], np.uint32))
_KS0 = int(_o0[0])
_KS1 = int(_o1[0])
_KS2 = _KS0 ^ _KS1 ^ 0x1BD11BDA
_LOG_RANGE = float(np.log(np.float32(RANGE_MAX + 1.0)))

_R, _C = 64, 128


def _tc_body(out_ref):
    ks = (jnp.uint32(_KS0), jnp.uint32(_KS1), jnp.uint32(_KS2))
    idx = (lax.broadcasted_iota(jnp.uint32, (_R, _C), 0) * jnp.uint32(_C)
           + lax.broadcasted_iota(jnp.uint32, (_R, _C), 1))
    x0 = jnp.full((_R, _C), jnp.uint32(_KS0), jnp.uint32)
    x1 = idx + ks[1]
    for i in range(5):
        for r in (_ROT_A if i % 2 == 0 else _ROT_B):
            x0 = x0 + x1
            x1 = (x1 << jnp.uint32(r)) | (x1 >> jnp.uint32(32 - r))
            x1 = x0 ^ x1
        x0 = x0 + ks[(i + 1) % 3]
        x1 = x1 + ks[(i + 2) % 3] + jnp.uint32(i + 1)
    bits = x0 ^ x1
    fb = (bits >> jnp.uint32(9)) | jnp.uint32(0x3F800000)
    u = lax.bitcast_convert_type(fb, jnp.float32) - jnp.float32(1.0)
    g = jnp.exp(u * jnp.float32(_LOG_RANGE))
    s = lax.convert_element_type(g, jnp.int32) - jnp.int32(1)
    out_ref[...] = jnp.minimum(jnp.maximum(s, jnp.int32(0)), jnp.int32(RANGE_MAX - 1))


def kernel(inputs):
    del inputs
    out = pl.pallas_call(
        _tc_body,
        out_shape=jax.ShapeDtypeStruct((_R, _C), jnp.int32),
    )()
    return out.reshape(NEG)


# speedup vs baseline: 26.9319x; 1.0051x over previous
import numpy as np
import jax
import jax.numpy as jnp
from jax import lax
from jax.experimental import pallas as pl

NEG = 8192
RANGE_MAX = 1000000

_ROT_A = (13, 15, 26, 6)
_ROT_B = (17, 29, 16, 24)


def _np_threefry2x32(ks0, ks1, x0, x1):
    x0 = x0.astype(np.uint32)
    x1 = x1.astype(np.uint32)
    ks = [np.uint32(ks0), np.uint32(ks1),
          np.uint32(np.uint32(ks0) ^ np.uint32(ks1) ^ np.uint32(0x1BD11BDA))]
    rot = [_ROT_A, _ROT_B]
    x0 = (x0 + ks[0]).astype(np.uint32)
    x1 = (x1 + ks[1]).astype(np.uint32)
    for i in range(5):
        for r in rot[i % 2]:
            x0 = (x0 + x1).astype(np.uint32)
            x1 = ((x1 << np.uint32(r)) | (x1 >> np.uint32(32 - r))).astype(np.uint32)
            x1 = (x0 ^ x1).astype(np.uint32)
        x0 = (x0 + ks[(i + 1) % 3]).astype(np.uint32)
        x1 = (x1 + ks[(i + 2) % 3] + np.uint32(i + 1)).astype(np.uint32)
    return x0, x1


_o0, _o1 = _np_threefry2x32(0, 0, np.array([0], np.uint32), np.array([1234], np.uint32))
_KS0 = int(_o0[0])
_KS1 = int(_o1[0])
_KS2 = _KS0 ^ _KS1 ^ 0x1BD11BDA
_LOG_RANGE = float(np.log(np.float32(RANGE_MAX + 1.0)))

_R, _C = 64, 128


def _tc_body(out_ref):
    ks = (jnp.uint32(_KS0), jnp.uint32(_KS1), jnp.uint32(_KS2))
    idx = (lax.broadcasted_iota(jnp.uint32, (_R, _C), 0) * jnp.uint32(_C)
           + lax.broadcasted_iota(jnp.uint32, (_R, _C), 1))
    x0 = jnp.full((_R, _C), jnp.uint32(_KS0), jnp.uint32)
    x1 = idx + ks[1]
    for i in range(5):
        for r in (_ROT_A if i % 2 == 0 else _ROT_B):
            x0 = x0 + x1
            x1 = (x1 << jnp.uint32(r)) | (x1 >> jnp.uint32(32 - r))
            x1 = x0 ^ x1
        x0 = x0 + ks[(i + 1) % 3]
        x1 = x1 + ks[(i + 2) % 3] + jnp.uint32(i + 1)
    bits = x0 ^ x1
    fb = (bits >> jnp.uint32(9)) | jnp.uint32(0x3F800000)
    u = lax.bitcast_convert_type(fb, jnp.float32) - jnp.float32(1.0)
    g = jnp.exp(u * jnp.float32(_LOG_RANGE))
    s = lax.convert_element_type(g, jnp.int32) - jnp.int32(1)
    out_ref[...] = jnp.maximum(s, jnp.int32(0))


def kernel(inputs):
    del inputs
    out = pl.pallas_call(
        _tc_body,
        out_shape=jax.ShapeDtypeStruct((_R, _C), jnp.int32),
    )()
    return out.reshape(NEG)
